# baseline (speedup 1.0000x reference)
import jax
import jax.numpy as jnp
from jax import lax
from jax.experimental import pallas as pl

NUM_CLUSTERS = 64
TOPK = 5
KMEANS_ITERS = 3
HEADS = 12
DIM_HEAD = 64

_NEG_NONMEMBER = -1e9
_NEG_INVALID = -2e9
_NEG_PICKED = -3e9

_F32 = jnp.float32
_BF16 = jnp.bfloat16


def _dot(a, b, precision=None):
    return lax.dot_general(a, b, ((((1,), (0,))), ((), ())),
                           preferred_element_type=_F32, precision=precision)


def _dotx(a, b):
    return _dot(a, b, precision=lax.Precision.HIGHEST)


def _fused_body(x_ref, wqkv_ref, wo_ref, b_ref, out_ref):
    X = x_ref[0]
    N, D = X.shape
    C = NUM_CLUSTERS
    DH = DIM_HEAD
    inner = HEADS * DH
    Xbf = X.astype(_BF16)
    XTbf = jnp.transpose(Xbf)

    tT = lax.broadcasted_iota(jnp.int32, (C, N), 1)
    cT = lax.broadcasted_iota(jnp.int32, (C, N), 0)
    validT = tT >= 1

    sel0 = (tT == cT + 1).astype(_F32)
    cen = _dotx(sel0, X)

    def kmeans_iter(_, carry):
        cen, _ = carry
        cnorm = jnp.sum(cen * cen, axis=1, keepdims=True)
        AT = _dot(cen.astype(_BF16), XTbf)
        scores = cnorm - 2.0 * AT
        minv = jnp.min(scores, axis=0, keepdims=True)
        assign = jnp.min(jnp.where(scores == minv, cT, C), axis=0,
                         keepdims=True)
        onehot = ((cT == assign) & validT).astype(_F32)
        counts = jnp.maximum(jnp.sum(onehot, axis=1, keepdims=True), 1.0)
        return _dot(onehot.astype(_BF16), Xbf) / counts, onehot

    cen, onehot = lax.fori_loop(
        0, KMEANS_ITERS, kmeans_iter,
        (cen, jnp.zeros((C, N), _F32)))

    sims = _dot(cen.astype(_BF16), XTbf)
    masked = jnp.where(onehot > 0.5, sims,
                       jnp.where(validT, _NEG_NONMEMBER, _NEG_INVALID))

    Wqkv = wqkv_ref[...]
    k_parts, v_parts = [], []
    for j in range(TOPK):
        mx = jnp.max(masked, axis=1, keepdims=True)
        selcol = jnp.min(jnp.where(masked == mx, tT, N), axis=1,
                         keepdims=True)
        selmask = tT == selcol
        xk_j = _dot(selmask.astype(_BF16), Xbf).astype(_BF16)
        k_parts.append(
            (_dot(xk_j, Wqkv[:, inner:2 * inner]) * (DH ** -0.5)).astype(_BF16))
        v_parts.append(_dot(xk_j, Wqkv[:, 2 * inner:]).astype(_BF16))
        masked = jnp.where(selmask, _NEG_PICKED, masked)

    kt = jnp.transpose(jnp.concatenate(k_parts, axis=0))
    v = jnp.concatenate(v_parts, axis=0)

    q = _dot(Xbf, Wqkv[:, :inner]).astype(_BF16)
    ys = []
    for h in range(HEADS):
        kt_h = kt[h * DH:(h + 1) * DH, :]
        v_h = v[:, h * DH:(h + 1) * DH]
        dots = _dot(q[:, h * DH:(h + 1) * DH], kt_h)
        m = jnp.max(dots, axis=1, keepdims=True)
        e = jnp.exp(dots - m)
        p = e * (1.0 / jnp.sum(e, axis=1, keepdims=True))
        ys.append(_dot(p.astype(_BF16), v_h).astype(_BF16))
    Yall = jnp.concatenate(ys, axis=1)
    out_ref[0] = _dot(Yall, wo_ref[...]) + b_ref[0]
    out_ref[0, 0:1, :] = x_ref[0, 0:1, :]


@jax.jit
def kernel(x, W_qkv, W_out, b_out):
    B, N, D = x.shape
    inner = HEADS * DIM_HEAD

    Wqkv_bf = W_qkv.astype(_BF16)
    Wo_bf = W_out.astype(_BF16)
    b2 = b_out.reshape(1, D)

    Y = pl.pallas_call(
        _fused_body,
        grid=(B,),
        in_specs=[
            pl.BlockSpec((1, N, D), lambda b: (b, 0, 0)),
            pl.BlockSpec((D, 3 * inner), lambda b: (0, 0)),
            pl.BlockSpec((inner, D), lambda b: (0, 0)),
            pl.BlockSpec((1, D), lambda b: (0, 0)),
        ],
        out_specs=pl.BlockSpec((1, N, D), lambda b: (b, 0, 0)),
        out_shape=jax.ShapeDtypeStruct((B, N, D), _F32),
    )(x, Wqkv_bf, Wo_bf, b2)
    return Y
